# Initial kernel scaffold; baseline (speedup 1.0000x reference)
import jax
import jax.numpy as jnp
from jax.experimental import pallas as pl


def kernel(x, edge_index, Wl1, bl1, Wr1, br1, att1, bias1, Wl2, bl2, Wr2, br2, att2, bias2):
    raise NotImplementedError("write your pallas kernel here")


# speedup vs baseline: 1.6870x; 1.6870x over previous
import functools

import jax
import jax.numpy as jnp
from jax import lax
from jax.experimental import pallas as pl
from jax.experimental.pallas import tpu as pltpu
from jax.experimental.pallas import tpu_sc as plsc

_L = 16
_N = 4
_E = 10

_O_WL1 = 0
_O_BL1 = 16
_O_WR1 = 24
_O_BR1 = 40
_O_ATT1 = 48
_O_BIAS1 = 56
_O_WL2 = 64
_O_BL2 = 80
_O_WR2 = 82
_O_BR2 = 98
_O_ATT2 = 100
_O_BIAS2 = 102
_P_LEN = 112

_NEG = -1e30


def _lrelu(v):
    return jnp.where(v > 0.0, v, 0.2 * v)


@functools.cache
def _build_sc_gnn():
    return functools.partial(
        pl.kernel,
        out_type=jax.ShapeDtypeStruct((2, _L), jnp.float32),
        mesh=plsc.VectorSubcoreMesh(
            core_axis_name="c", subcore_axis_name="s", num_cores=2, num_subcores=16
        ),
        scratch_types=[
            pltpu.VMEM((2, _L), jnp.float32),
            pltpu.VMEM((2, _L), jnp.int32),
            pltpu.VMEM((_P_LEN,), jnp.float32),
            pltpu.VMEM((16, _L), jnp.float32),
            pltpu.VMEM((16, _L), jnp.float32),
            pltpu.VMEM((2, _L), jnp.float32),
        ],
        compiler_params=pltpu.CompilerParams(needs_layout_passes=False),
    )(_sc_gnn_body)


def _sc_gnn_body(x_hbm, ei_hbm, par_hbm, out_hbm, x_v, ei_v, par_v, f1_v, f2_v, out_v):
    cid = lax.axis_index("c")
    sid = lax.axis_index("s")

    @pl.when(jnp.logical_and(cid == 0, sid == 0))
    def _body():
        pltpu.sync_copy(x_hbm, x_v)
        pltpu.sync_copy(ei_hbm, ei_v)
        pltpu.sync_copy(par_hbm, par_v)

        src = ei_v[0, :]
        dst = ei_v[1, :]
        xc0 = x_v[0, :]
        xc1 = x_v[1, :]
        lane = lax.iota(jnp.int32, _L)

        chunks = [par_v[pl.ds(16 * j, 16)] for j in range(_P_LEN // 16)]

        def p(i):
            return chunks[i // 16][i % 16]

        def splat(i):
            return jnp.full((_L,), i, jnp.int32)

        for k in range(8):
            f1_v[k, :] = xc0 * p(_O_WL1 + 2 * k) + xc1 * p(_O_WL1 + 2 * k + 1) + p(_O_BL1 + k)
            f1_v[8 + k, :] = xc0 * p(_O_WR1 + 2 * k) + xc1 * p(_O_WR1 + 2 * k + 1) + p(_O_BR1 + k)

        xj = [plsc.load_gather(f1_v, [splat(c), src]) for c in range(8)]
        xi = [plsc.load_gather(f1_v, [splat(8 + c), dst]) for c in range(8)]
        e1 = [_lrelu(xi[c] + xj[c]) for c in range(8)]
        alpha = []
        for h in range(2):
            a = e1[4 * h] * p(_O_ATT1 + 4 * h)
            for c in range(1, 4):
                a = a + e1[4 * h + c] * p(_O_ATT1 + 4 * h + c)
            alpha.append(a)

        agg = [jnp.zeros((_L,), jnp.float32) for _ in range(8)]
        for n in range(_N):
            m = dst == n
            ln = lane == n
            for h in range(2):
                am = jnp.max(jnp.where(m, alpha[h], _NEG))
                ex = jnp.where(m, jnp.exp(alpha[h] - am), 0.0)
                coef = ex / (jnp.sum(ex) + 1e-16)
                for c in range(4):
                    k = 4 * h + c
                    agg[k] = jnp.where(ln, jnp.sum(coef * xj[k]), agg[k])

        h_rows = []
        for k in range(8):
            v = agg[k] + p(_O_BIAS1 + k)
            v = jnp.where(v > 0.0, v, jnp.exp(v) - 1.0)
            h_rows.append(v)

        for k in range(2):
            al = h_rows[0] * p(_O_WL2 + 8 * k)
            ar = h_rows[0] * p(_O_WR2 + 8 * k)
            for c in range(1, 8):
                al = al + h_rows[c] * p(_O_WL2 + 8 * k + c)
                ar = ar + h_rows[c] * p(_O_WR2 + 8 * k + c)
            f2_v[8 + k, :] = al + p(_O_BL2 + k)
            f2_v[10 + k, :] = ar + p(_O_BR2 + k)

        xj2 = [plsc.load_gather(f2_v, [splat(8 + k), src]) for k in range(2)]
        xi2 = [plsc.load_gather(f2_v, [splat(10 + k), dst]) for k in range(2)]
        e2 = [_lrelu(xi2[k] + xj2[k]) for k in range(2)]
        alpha2 = e2[0] * p(_O_ATT2) + e2[1] * p(_O_ATT2 + 1)

        out_rows = [jnp.zeros((_L,), jnp.float32) for _ in range(2)]
        for n in range(_N):
            m = dst == n
            ln = lane == n
            am = jnp.max(jnp.where(m, alpha2, _NEG))
            ex = jnp.where(m, jnp.exp(alpha2 - am), 0.0)
            coef = ex / (jnp.sum(ex) + 1e-16)
            for k in range(2):
                out_rows[k] = jnp.where(ln, jnp.sum(coef * xj2[k]), out_rows[k])
        for k in range(2):
            out_v[k, :] = out_rows[k] + p(_O_BIAS2 + k)

        pltpu.sync_copy(out_v, out_hbm)


def kernel(x, edge_index, Wl1, bl1, Wr1, br1, att1, bias1, Wl2, bl2, Wr2, br2, att2, bias2):
    loop = jnp.arange(_N, dtype=edge_index.dtype)
    src = jnp.concatenate([edge_index[0], loop, jnp.zeros((_L - _E,), edge_index.dtype)])
    dst = jnp.concatenate([edge_index[1], loop, jnp.full((_L - _E,), _L - 1, edge_index.dtype)])
    ei = jnp.stack([src, dst]).astype(jnp.int32)
    xt = jnp.zeros((2, _L), jnp.float32).at[:, :_N].set(x.T)
    params = jnp.concatenate([
        Wl1.ravel(), bl1, Wr1.ravel(), br1, att1.ravel(), bias1,
        Wl2.ravel(), bl2, Wr2.ravel(), br2, att2.ravel(), bias2,
        jnp.zeros((_P_LEN - 104,), jnp.float32),
    ])
    out_pad = _build_sc_gnn()(xt, ei, params)
    return out_pad[:, :_N].T


# speedup vs baseline: 1.9231x; 1.1400x over previous
import functools

import jax
import jax.numpy as jnp
from jax import lax
from jax.experimental import pallas as pl
from jax.experimental.pallas import tpu as pltpu
from jax.experimental.pallas import tpu_sc as plsc

_L = 16
_N = 4
_E = 10

_O_X = 0
_O_W1C0 = 16
_O_W1C1 = 32
_O_B1 = 48
_O_ATT1 = 64
_O_BIAS1 = 72
_O_WL2 = 80
_O_WR2 = 96
_O_BL2 = 112
_O_BR2 = 114
_O_ATT2 = 116
_O_BIAS2 = 118
_O_SRC = 128
_O_DST = 144
_P_LEN = 160

_NEG = -1e30


def _lrelu(v):
    return jnp.where(v > 0.0, v, 0.2 * v)


@functools.cache
def _build_sc_gnn():
    return functools.partial(
        pl.kernel,
        out_type=jax.ShapeDtypeStruct((_N, 2), jnp.float32),
        mesh=plsc.VectorSubcoreMesh(
            core_axis_name="c", subcore_axis_name="s", num_cores=2, num_subcores=16
        ),
        scratch_types=[
            pltpu.VMEM((_P_LEN,), jnp.float32),
            pltpu.VMEM((16, _L), jnp.float32),
            pltpu.VMEM((16, _L), jnp.float32),
            pltpu.VMEM((_N, 2), jnp.float32),
        ],
        compiler_params=pltpu.CompilerParams(needs_layout_passes=False),
    )(_sc_gnn_body)


def _sc_gnn_body(par_hbm, out_hbm, par_v, feat_v, seg_v, out_v):
    cid = lax.axis_index("c")
    sid = lax.axis_index("s")

    @pl.when(jnp.logical_and(cid == 0, sid == 0))
    def _body():
        pltpu.sync_copy(par_hbm, par_v)

        lane = lax.iota(jnp.int32, _L)
        valid = lane < _E
        zeros = jnp.zeros((_L,), jnp.float32)

        chunks = [par_v[pl.ds(16 * j, 16)] for j in range(_P_LEN // 16)]

        def p(i):
            return chunks[i // 16][i % 16]

        def splat(i):
            return jnp.full((_L,), i, jnp.int32)

        src = plsc.bitcast(chunks[_O_SRC // 16], jnp.int32)
        dst = plsc.bitcast(chunks[_O_DST // 16], jnp.int32)
        w1c0 = chunks[_O_W1C0 // 16]
        w1c1 = chunks[_O_W1C1 // 16]
        b1 = chunks[_O_B1 // 16]

        for n in range(_N):
            feat_v[n, :] = p(_O_X + 2 * n) * w1c0 + p(_O_X + 2 * n + 1) * w1c1 + b1
        feat_v[15, :] = zeros

        xj = [plsc.load_gather(feat_v, [src, splat(c)]) for c in range(8)]
        xi = [plsc.load_gather(feat_v, [dst, splat(8 + c)]) for c in range(8)]
        e1 = [_lrelu(xi[c] + xj[c]) for c in range(8)]
        alpha = []
        for h in range(2):
            a = e1[4 * h] * p(_O_ATT1 + 4 * h)
            for c in range(1, 4):
                a = a + e1[4 * h + c] * p(_O_ATT1 + 4 * h + c)
            alpha.append(jnp.where(valid, a, _NEG))

        for r in range(13):
            seg_v[r, :] = zeros
        coef = []
        for h in range(2):
            ex = jnp.exp(alpha[h] - jnp.max(alpha[h]))
            plsc.addupdate_scatter(seg_v, [splat(h), dst], ex)
            den = plsc.load_gather(seg_v, [splat(h), dst])
            coef.append(ex / (den + 1e-16))
        for k in range(8):
            plsc.addupdate_scatter(seg_v, [splat(2 + k), dst], coef[k // 4] * xj[k])

        hr = []
        for k in range(8):
            v = seg_v[2 + k, :] + p(_O_BIAS1 + k)
            hr.append(jnp.where(v > 0.0, v, jnp.exp(v) - 1.0))

        for k in range(2):
            al = hr[0] * p(_O_WL2 + 8 * k)
            ar = hr[0] * p(_O_WR2 + 8 * k)
            for c in range(1, 8):
                al = al + hr[c] * p(_O_WL2 + 8 * k + c)
                ar = ar + hr[c] * p(_O_WR2 + 8 * k + c)
            feat_v[4 + k, :] = al + p(_O_BL2 + k)
            feat_v[6 + k, :] = ar + p(_O_BR2 + k)

        xj2 = [plsc.load_gather(feat_v, [splat(4 + k), src]) for k in range(2)]
        xi2 = [plsc.load_gather(feat_v, [splat(6 + k), dst]) for k in range(2)]
        e2 = [_lrelu(xi2[k] + xj2[k]) for k in range(2)]
        alpha2 = jnp.where(
            valid, e2[0] * p(_O_ATT2) + e2[1] * p(_O_ATT2 + 1), _NEG
        )
        ex2 = jnp.exp(alpha2 - jnp.max(alpha2))
        plsc.addupdate_scatter(seg_v, [splat(10), dst], ex2)
        den2 = plsc.load_gather(seg_v, [splat(10), dst])
        coef2 = ex2 / (den2 + 1e-16)
        for k in range(2):
            plsc.addupdate_scatter(seg_v, [splat(11 + k), dst], coef2 * xj2[k])

        nmask = lane < _N
        for k in range(2):
            col = seg_v[11 + k, :] + p(_O_BIAS2 + k)
            plsc.store_scatter(out_v, [lane & 3, splat(k)], col, mask=nmask)
        pltpu.sync_copy(out_v, out_hbm)


def kernel(x, edge_index, Wl1, bl1, Wr1, br1, att1, bias1, Wl2, bl2, Wr2, br2, att2, bias2):
    loop = jnp.arange(_N, dtype=edge_index.dtype)
    src = jnp.concatenate([edge_index[0], loop, jnp.zeros((_L - _E,), edge_index.dtype)])
    dst = jnp.concatenate([edge_index[1], loop, jnp.full((_L - _E,), _L - 1, edge_index.dtype)])
    f32 = jnp.float32
    params = jnp.concatenate([
        x.ravel(), jnp.zeros((8,), f32),
        Wl1[:, 0], Wr1[:, 0],
        Wl1[:, 1], Wr1[:, 1],
        bl1, br1,
        att1.ravel(), bias1,
        Wl2.ravel(), Wr2.ravel(),
        bl2, br2, att2.ravel(), bias2, jnp.zeros((8,), f32),
        lax.bitcast_convert_type(src.astype(jnp.int32), f32),
        lax.bitcast_convert_type(dst.astype(jnp.int32), f32),
    ])
    return _build_sc_gnn()(params)


# speedup vs baseline: 2.0328x; 1.0571x over previous
import functools

import jax
import jax.numpy as jnp
from jax import lax
from jax.experimental import pallas as pl
from jax.experimental.pallas import tpu as pltpu
from jax.experimental.pallas import tpu_sc as plsc

_L = 16
_N = 4
_E = 10

_O_X = 0
_O_W1C0 = 16
_O_W1C1 = 32
_O_B1 = 48
_O_ATT1 = 64
_O_BIAS1 = 72
_O_WL2 = 80
_O_WR2 = 96
_O_BL2 = 112
_O_BR2 = 114
_O_ATT2 = 116
_O_BIAS2 = 118
_O_SRC = 128
_O_DST = 144
_P_LEN = 160

_NEG = -1e30


def _lrelu(v):
    return jnp.where(v > 0.0, v, 0.2 * v)


@functools.cache
def _build_sc_gnn():
    return functools.partial(
        pl.kernel,
        out_type=jax.ShapeDtypeStruct((_N, 2), jnp.float32),
        mesh=plsc.VectorSubcoreMesh(
            core_axis_name="c", subcore_axis_name="s", num_cores=1, num_subcores=16
        ),
        scratch_types=[
            pltpu.VMEM((_P_LEN,), jnp.float32),
            pltpu.VMEM((16, _L), jnp.float32),
            pltpu.VMEM((16, _L), jnp.float32),
            pltpu.VMEM((_N, 2), jnp.float32),
        ],
        compiler_params=pltpu.CompilerParams(needs_layout_passes=False),
    )(_sc_gnn_body)


def _sc_gnn_body(par_hbm, out_hbm, par_v, feat_v, seg_v, out_v):
    cid = lax.axis_index("c")
    sid = lax.axis_index("s")

    @pl.when(jnp.logical_and(cid == 0, sid == 0))
    def _body():
        pltpu.sync_copy(par_hbm, par_v)

        lane = lax.iota(jnp.int32, _L)
        valid = lane < _E
        zeros = jnp.zeros((_L,), jnp.float32)

        chunks = [par_v[pl.ds(16 * j, 16)] for j in range(_P_LEN // 16)]

        def p(i):
            return chunks[i // 16][i % 16]

        def splat(i):
            return jnp.full((_L,), i, jnp.int32)

        src = plsc.bitcast(chunks[_O_SRC // 16], jnp.int32)
        dst = plsc.bitcast(chunks[_O_DST // 16], jnp.int32)
        w1c0 = chunks[_O_W1C0 // 16]
        w1c1 = chunks[_O_W1C1 // 16]
        b1 = chunks[_O_B1 // 16]

        for n in range(_N):
            feat_v[n, :] = p(_O_X + 2 * n) * w1c0 + p(_O_X + 2 * n + 1) * w1c1 + b1
        feat_v[15, :] = zeros

        xj = [plsc.load_gather(feat_v, [src, splat(c)]) for c in range(8)]
        xi = [plsc.load_gather(feat_v, [dst, splat(8 + c)]) for c in range(8)]
        e1 = [_lrelu(xi[c] + xj[c]) for c in range(8)]
        alpha = []
        for h in range(2):
            a = e1[4 * h] * p(_O_ATT1 + 4 * h)
            for c in range(1, 4):
                a = a + e1[4 * h + c] * p(_O_ATT1 + 4 * h + c)
            alpha.append(jnp.where(valid, a, _NEG))

        for r in range(13):
            seg_v[r, :] = zeros
        coef = []
        for h in range(2):
            ex = jnp.exp(alpha[h] - jnp.max(alpha[h]))
            plsc.addupdate_scatter(seg_v, [splat(h), dst], ex)
            den = plsc.load_gather(seg_v, [splat(h), dst])
            coef.append(ex / (den + 1e-16))
        for k in range(8):
            plsc.addupdate_scatter(seg_v, [splat(2 + k), dst], coef[k // 4] * xj[k])

        hr = []
        for k in range(8):
            v = seg_v[2 + k, :] + p(_O_BIAS1 + k)
            hr.append(jnp.where(v > 0.0, v, jnp.exp(v) - 1.0))

        for k in range(2):
            al = hr[0] * p(_O_WL2 + 8 * k)
            ar = hr[0] * p(_O_WR2 + 8 * k)
            for c in range(1, 8):
                al = al + hr[c] * p(_O_WL2 + 8 * k + c)
                ar = ar + hr[c] * p(_O_WR2 + 8 * k + c)
            feat_v[4 + k, :] = al + p(_O_BL2 + k)
            feat_v[6 + k, :] = ar + p(_O_BR2 + k)

        xj2 = [plsc.load_gather(feat_v, [splat(4 + k), src]) for k in range(2)]
        xi2 = [plsc.load_gather(feat_v, [splat(6 + k), dst]) for k in range(2)]
        e2 = [_lrelu(xi2[k] + xj2[k]) for k in range(2)]
        alpha2 = jnp.where(
            valid, e2[0] * p(_O_ATT2) + e2[1] * p(_O_ATT2 + 1), _NEG
        )
        ex2 = jnp.exp(alpha2 - jnp.max(alpha2))
        plsc.addupdate_scatter(seg_v, [splat(10), dst], ex2)
        den2 = plsc.load_gather(seg_v, [splat(10), dst])
        coef2 = ex2 / (den2 + 1e-16)
        for k in range(2):
            plsc.addupdate_scatter(seg_v, [splat(11 + k), dst], coef2 * xj2[k])

        nmask = lane < _N
        for k in range(2):
            col = seg_v[11 + k, :] + p(_O_BIAS2 + k)
            plsc.store_scatter(out_v, [lane & 3, splat(k)], col, mask=nmask)
        pltpu.sync_copy(out_v, out_hbm)


def kernel(x, edge_index, Wl1, bl1, Wr1, br1, att1, bias1, Wl2, bl2, Wr2, br2, att2, bias2):
    loop = jnp.arange(_N, dtype=edge_index.dtype)
    src = jnp.concatenate([edge_index[0], loop, jnp.zeros((_L - _E,), edge_index.dtype)])
    dst = jnp.concatenate([edge_index[1], loop, jnp.full((_L - _E,), _L - 1, edge_index.dtype)])
    f32 = jnp.float32
    params = jnp.concatenate([
        x.ravel(), jnp.zeros((8,), f32),
        Wl1[:, 0], Wr1[:, 0],
        Wl1[:, 1], Wr1[:, 1],
        bl1, br1,
        att1.ravel(), bias1,
        Wl2.ravel(), Wr2.ravel(),
        bl2, br2, att2.ravel(), bias2, jnp.zeros((8,), f32),
        lax.bitcast_convert_type(src.astype(jnp.int32), f32),
        lax.bitcast_convert_type(dst.astype(jnp.int32), f32),
    ])
    return _build_sc_gnn()(params)


# speedup vs baseline: 2.0440x; 1.0055x over previous
import functools

import jax
import jax.numpy as jnp
from jax import lax
from jax.experimental import pallas as pl
from jax.experimental.pallas import tpu as pltpu
from jax.experimental.pallas import tpu_sc as plsc

_L = 16
_N = 4
_E = 10

_O_X = 0
_O_W1C0 = 16
_O_W1C1 = 32
_O_B1 = 48
_O_ATT1 = 64
_O_BIAS1 = 72
_O_WL2 = 80
_O_WR2 = 96
_O_BL2 = 112
_O_BR2 = 114
_O_ATT2 = 116
_O_BIAS2 = 118
_O_SRC = 128
_O_DST = 144
_P_LEN = 160

_NEG = -1e30


def _lrelu(v):
    return jnp.where(v > 0.0, v, 0.2 * v)


@functools.cache
def _build_sc_gnn():
    return functools.partial(
        pl.kernel,
        out_type=jax.ShapeDtypeStruct((_N, 2), jnp.float32),
        mesh=plsc.VectorSubcoreMesh(
            core_axis_name="c", subcore_axis_name="s", num_cores=1, num_subcores=1
        ),
        scratch_types=[
            pltpu.VMEM((_P_LEN,), jnp.float32),
            pltpu.VMEM((16, _L), jnp.float32),
            pltpu.VMEM((16, _L), jnp.float32),
            pltpu.VMEM((_N, 2), jnp.float32),
        ],
        compiler_params=pltpu.CompilerParams(needs_layout_passes=False),
    )(_sc_gnn_body)


def _sc_gnn_body(par_hbm, out_hbm, par_v, feat_v, seg_v, out_v):
    cid = lax.axis_index("c")
    sid = lax.axis_index("s")

    @pl.when(jnp.logical_and(cid == 0, sid == 0))
    def _body():
        pltpu.sync_copy(par_hbm, par_v)

        lane = lax.iota(jnp.int32, _L)
        valid = lane < _E
        zeros = jnp.zeros((_L,), jnp.float32)

        chunks = [par_v[pl.ds(16 * j, 16)] for j in range(_P_LEN // 16)]

        def p(i):
            return chunks[i // 16][i % 16]

        def splat(i):
            return jnp.full((_L,), i, jnp.int32)

        src = plsc.bitcast(chunks[_O_SRC // 16], jnp.int32)
        dst = plsc.bitcast(chunks[_O_DST // 16], jnp.int32)
        w1c0 = chunks[_O_W1C0 // 16]
        w1c1 = chunks[_O_W1C1 // 16]
        b1 = chunks[_O_B1 // 16]

        for n in range(_N):
            feat_v[n, :] = p(_O_X + 2 * n) * w1c0 + p(_O_X + 2 * n + 1) * w1c1 + b1
        feat_v[15, :] = zeros

        xj = [plsc.load_gather(feat_v, [src, splat(c)]) for c in range(8)]
        xi = [plsc.load_gather(feat_v, [dst, splat(8 + c)]) for c in range(8)]
        e1 = [_lrelu(xi[c] + xj[c]) for c in range(8)]
        alpha = []
        for h in range(2):
            a = e1[4 * h] * p(_O_ATT1 + 4 * h)
            for c in range(1, 4):
                a = a + e1[4 * h + c] * p(_O_ATT1 + 4 * h + c)
            alpha.append(jnp.where(valid, a, _NEG))

        for r in range(13):
            seg_v[r, :] = zeros
        coef = []
        for h in range(2):
            ex = jnp.exp(alpha[h] - jnp.max(alpha[h]))
            plsc.addupdate_scatter(seg_v, [splat(h), dst], ex)
            den = plsc.load_gather(seg_v, [splat(h), dst])
            coef.append(ex / (den + 1e-16))
        for k in range(8):
            plsc.addupdate_scatter(seg_v, [splat(2 + k), dst], coef[k // 4] * xj[k])

        hr = []
        for k in range(8):
            v = seg_v[2 + k, :] + p(_O_BIAS1 + k)
            hr.append(jnp.where(v > 0.0, v, jnp.exp(v) - 1.0))

        for k in range(2):
            al = hr[0] * p(_O_WL2 + 8 * k)
            ar = hr[0] * p(_O_WR2 + 8 * k)
            for c in range(1, 8):
                al = al + hr[c] * p(_O_WL2 + 8 * k + c)
                ar = ar + hr[c] * p(_O_WR2 + 8 * k + c)
            feat_v[4 + k, :] = al + p(_O_BL2 + k)
            feat_v[6 + k, :] = ar + p(_O_BR2 + k)

        xj2 = [plsc.load_gather(feat_v, [splat(4 + k), src]) for k in range(2)]
        xi2 = [plsc.load_gather(feat_v, [splat(6 + k), dst]) for k in range(2)]
        e2 = [_lrelu(xi2[k] + xj2[k]) for k in range(2)]
        alpha2 = jnp.where(
            valid, e2[0] * p(_O_ATT2) + e2[1] * p(_O_ATT2 + 1), _NEG
        )
        ex2 = jnp.exp(alpha2 - jnp.max(alpha2))
        plsc.addupdate_scatter(seg_v, [splat(10), dst], ex2)
        den2 = plsc.load_gather(seg_v, [splat(10), dst])
        coef2 = ex2 / (den2 + 1e-16)
        for k in range(2):
            plsc.addupdate_scatter(seg_v, [splat(11 + k), dst], coef2 * xj2[k])

        nmask = lane < _N
        for k in range(2):
            col = seg_v[11 + k, :] + p(_O_BIAS2 + k)
            plsc.store_scatter(out_v, [lane & 3, splat(k)], col, mask=nmask)
        pltpu.sync_copy(out_v, out_hbm)


def kernel(x, edge_index, Wl1, bl1, Wr1, br1, att1, bias1, Wl2, bl2, Wr2, br2, att2, bias2):
    loop = jnp.arange(_N, dtype=edge_index.dtype)
    src = jnp.concatenate([edge_index[0], loop, jnp.zeros((_L - _E,), edge_index.dtype)])
    dst = jnp.concatenate([edge_index[1], loop, jnp.full((_L - _E,), _L - 1, edge_index.dtype)])
    f32 = jnp.float32
    params = jnp.concatenate([
        x.ravel(), jnp.zeros((8,), f32),
        Wl1[:, 0], Wr1[:, 0],
        Wl1[:, 1], Wr1[:, 1],
        bl1, br1,
        att1.ravel(), bias1,
        Wl2.ravel(), Wr2.ravel(),
        bl2, br2, att2.ravel(), bias2, jnp.zeros((8,), f32),
        lax.bitcast_convert_type(src.astype(jnp.int32), f32),
        lax.bitcast_convert_type(dst.astype(jnp.int32), f32),
    ])
    return _build_sc_gnn()(params)
